# baseline (speedup 1.0000x reference)
import functools

import jax
import jax.numpy as jnp
from jax import lax
from jax.experimental import pallas as pl
from jax.experimental.pallas import tpu as pltpu
from jax.experimental.pallas import tpu_sc as plsc

L = 16
NC = 2
NS = 16
NW = NC * NS
CH = 400
GW = 80
NG = CH // GW


def _sc_edge_kernel(nchunk):
    mesh = plsc.VectorSubcoreMesh(core_axis_name="c", subcore_axis_name="s")
    per = nchunk * CH

    @functools.partial(
        pl.kernel,
        mesh=mesh,
        compiler_params=pltpu.CompilerParams(use_tc_tiling_on_sc=False),
        out_type=jax.ShapeDtypeStruct((NC, L), jnp.float32),
        scratch_types=[
            pltpu.VMEM((4, CH), jnp.int32),
            pltpu.VMEM((4, CH), jnp.int32),
            pltpu.VMEM((4, CH), jnp.float32),
            pltpu.VMEM((4, CH), jnp.float32),
            pltpu.VMEM((2, CH, L), jnp.float32),
            pltpu.VMEM((2, CH, L), jnp.float32),
            pltpu.VMEM((L,), jnp.float32),
            pltpu.VMEM((L,), jnp.float32),
            pltpu.VMEM_SHARED((NS, L), jnp.float32),
            pltpu.VMEM_SHARED((3200 * NS, L), jnp.float32),
            pltpu.SemaphoreType.DMA,
            pltpu.SemaphoreType.DMA,
            pltpu.SemaphoreType.DMA,
            pltpu.SemaphoreType.DMA,
            pltpu.SemaphoreType.DMA,
            pltpu.SemaphoreType.DMA,
        ],
    )
    def k(table_hbm, row_hbm, col_hbm, wp_hbm, wn_hbm, num_out,
          idx_a, idx_b, wp_v, wn_v, rows_a, rows_b,
          acc_v, tmp_v, shared_num, shared_tab,
          sem_l0, sem_l1, sem_l2, sem_l3, sem_g0, sem_g1):
        c = lax.axis_index("c")
        s = lax.axis_index("s")
        wid = s * NC + c
        ebase = wid * per
        sem_l = (sem_l0, sem_l1, sem_l2, sem_l3)
        sem_g = (sem_g0, sem_g1)

        trows = shared_tab.shape[0] // NS
        toff = pl.multiple_of(s * trows, 8)
        pltpu.sync_copy(table_hbm.at[pl.ds(toff, trows)],
                        shared_tab.at[pl.ds(toff, trows)])
        plsc.subcore_barrier()

        def fire_lin(g, b):
            off = pl.multiple_of(ebase + g * CH, 8)
            pltpu.async_copy(row_hbm.at[pl.ds(off, CH)], idx_a.at[b], sem_l[b])
            pltpu.async_copy(col_hbm.at[pl.ds(off, CH)], idx_b.at[b], sem_l[b])
            pltpu.async_copy(wp_hbm.at[pl.ds(off, CH)], wp_v.at[b], sem_l[b])
            pltpu.async_copy(wn_hbm.at[pl.ds(off, CH)], wn_v.at[b], sem_l[b])

        def wait_lin(b):
            pltpu.make_async_copy(row_hbm.at[pl.ds(0, CH)], idx_a.at[b],
                                  sem_l[b]).wait()
            pltpu.make_async_copy(col_hbm.at[pl.ds(0, CH)], idx_b.at[b],
                                  sem_l[b]).wait()
            pltpu.make_async_copy(wp_hbm.at[pl.ds(0, CH)], wp_v.at[b],
                                  sem_l[b]).wait()
            pltpu.make_async_copy(wn_hbm.at[pl.ds(0, CH)], wn_v.at[b],
                                  sem_l[b]).wait()

        def fire_gath(lb, rb):
            for j in range(NG):
                sl = pl.ds(j * GW, GW)
                pltpu.async_copy(shared_tab.at[idx_a.at[lb].at[sl]],
                                 rows_a.at[rb].at[sl], sem_g[rb])
                pltpu.async_copy(shared_tab.at[idx_b.at[lb].at[sl]],
                                 rows_b.at[rb].at[sl], sem_g[rb])

        def wait_gath(rb):
            for j in range(NG):
                sl = pl.ds(j * GW, GW)
                pltpu.make_async_copy(table_hbm.at[pl.ds(0, GW)],
                                      rows_a.at[rb].at[sl], sem_g[rb]).wait()
                pltpu.make_async_copy(table_hbm.at[pl.ds(0, GW)],
                                      rows_b.at[rb].at[sl], sem_g[rb]).wait()

        def compute(lb, rb, acc):
            ra = rows_a.at[rb]
            rbv = rows_b.at[rb]
            wpr = wp_v.at[lb]
            wnr = wn_v.at[lb]

            def group_body(gi, acc):
                eb = gi * L
                wp16 = wpr[pl.ds(eb, L)]
                wd16 = wnr[pl.ds(eb, L)] - wp16
                for j in range(L):
                    a = ra[eb + j]
                    b = rbv[eb + j]
                    wpe = jnp.full((L,), wp16[j], jnp.float32)
                    wde = jnp.full((L,), wd16[j], jnp.float32)
                    acc = acc + a * (wpe * a + wde * b)
                return acc

            return plsc.parallel_loop(0, CH // L, 1, unroll=2,
                                      carry=acc)(group_body)

        nq = nchunk // 4

        fire_lin(0, 0)
        fire_lin(1, 1)
        wait_lin(0)
        fire_gath(0, 0)

        def quad_body(i, acc):
            g0 = 4 * i
            wait_lin(1)
            fire_gath(1, 1)
            fire_lin(g0 + 2, 2)
            wait_gath(0)
            acc = compute(0, 0, acc)
            wait_lin(2)
            fire_gath(2, 0)
            fire_lin(g0 + 3, 3)
            wait_gath(1)
            acc = compute(1, 1, acc)
            wait_lin(3)
            fire_gath(3, 1)
            fire_lin(g0 + 4, 0)
            wait_gath(0)
            acc = compute(2, 0, acc)
            wait_lin(0)
            fire_gath(0, 0)
            fire_lin(jnp.minimum(g0 + 5, nchunk - 1), 1)
            wait_gath(1)
            acc = compute(3, 1, acc)
            return acc

        num_acc = lax.fori_loop(0, nq, quad_body, jnp.zeros((L,), jnp.float32))

        wait_gath(0)
        num_acc = compute(0, 0, num_acc)
        wait_lin(1)

        acc_v[...] = num_acc
        pltpu.sync_copy(acc_v, shared_num.at[s])
        plsc.subcore_barrier()

        @pl.when(s == 0)
        def _():
            num_tot = jnp.zeros((L,), jnp.float32)
            for j in range(NS):
                pltpu.sync_copy(shared_num.at[j], tmp_v)
                num_tot = num_tot + tmp_v[...]
            acc_v[...] = num_tot
            pltpu.sync_copy(acc_v, num_out.at[c])

    return k


def _split_body(ei_ref, row_ref, col_ref):
    x = ei_ref[...]
    row_ref[...] = jnp.reshape(x[0:1, :], row_ref.shape)
    col_ref[...] = jnp.reshape(x[1:2, :], col_ref.shape)


def _split_edges(edge_index, e):
    rows8 = -(-(e // 128) // 8) * 8
    br = 504
    grid = -(-rows8 // br)
    rows8 = grid * br
    bl = br * 128
    rows2, cols2 = pl.pallas_call(
        _split_body,
        grid=(grid,),
        in_specs=[pl.BlockSpec((2, bl), lambda i: (0, i))],
        out_specs=[pl.BlockSpec((br, 128), lambda i: (i, 0)),
                   pl.BlockSpec((br, 128), lambda i: (i, 0))],
        out_shape=[jax.ShapeDtypeStruct((rows8, 128), jnp.int32),
                   jax.ShapeDtypeStruct((rows8, 128), jnp.int32)],
    )(edge_index)
    return rows2.reshape(-1), cols2.reshape(-1)


def _flatten_body(pt_ref, tab_ref, den_ref, *, bc, n):
    i = pl.program_id(0)
    x = pt_ref[...]
    kk = x.shape[0]
    colmask = (i * bc + lax.broadcasted_iota(jnp.int32, (kk, bc), 1)) < n
    xm = jnp.where(colmask, x, 0.0)
    xt = jnp.transpose(xm)
    xt3 = jnp.reshape(xt, (bc // 8, 8, kk))
    z = jnp.zeros((bc // 8, L - kk), jnp.float32)
    pieces = []
    for t in range(8):
        pieces.append(xt3[:, t, :])
        pieces.append(z)
    tab_ref[...] = jnp.concatenate(pieces, axis=1)
    d = jnp.sum(xm * xm, axis=1)
    d16 = jnp.concatenate([d, jnp.zeros((L - kk,), jnp.float32)])

    @pl.when(i == 0)
    def _():
        den_ref[...] = jnp.zeros((1, L), jnp.float32)

    den_ref[...] += jnp.reshape(d16, (1, L))


def _flatten_prob(prob):
    n, kk = prob.shape
    bc = 3200
    grid = -(-n // bc)
    n_pad = grid * bc
    pt = prob.T
    tab, den = pl.pallas_call(
        lambda a, b, c: _flatten_body(a, b, c, bc=bc, n=n),
        grid=(grid,),
        in_specs=[pl.BlockSpec((kk, bc), lambda i: (0, i))],
        out_specs=[pl.BlockSpec((bc * L // 128, 128), lambda i: (i, 0)),
                   pl.BlockSpec((1, L), lambda i: (0, 0))],
        out_shape=[jax.ShapeDtypeStruct((n_pad * L // 128, 128), jnp.float32),
                   jax.ShapeDtypeStruct((1, L), jnp.float32)],
    )(pt)
    return tab.reshape(n_pad, L), den


def _combine_body(num_ref, den_ref, out_ref):
    num = jnp.sum(num_ref[...], axis=0, keepdims=True)
    den = den_ref[...] + 1.0
    out_ref[...] = jnp.full((1, L), jnp.sum(num / den), jnp.float32)


def kernel(prob, edge_index, w_p, w_n):
    e = w_p.shape[0]
    nchunk = e // (NW * CH)
    assert nchunk * CH * NW == e and nchunk % 4 == 1

    row, col = _split_edges(edge_index, e)
    table, den = _flatten_prob(prob)

    num_parts = _sc_edge_kernel(nchunk)(table, row, col, w_p, w_n)

    out = pl.pallas_call(
        _combine_body,
        out_shape=jax.ShapeDtypeStruct((1, L), jnp.float32),
    )(num_parts, den)
    return out[0, 0:1]


# speedup vs baseline: 1.0646x; 1.0646x over previous
import functools

import jax
import jax.numpy as jnp
from jax import lax
from jax.experimental import pallas as pl
from jax.experimental.pallas import tpu as pltpu
from jax.experimental.pallas import tpu_sc as plsc

L = 16
NC = 2
NS = 16
NW = NC * NS
CH = 400
GW = 80
NG = CH // GW


def _sc_edge_kernel(nchunk):
    mesh = plsc.VectorSubcoreMesh(core_axis_name="c", subcore_axis_name="s")
    per = nchunk * CH

    @functools.partial(
        pl.kernel,
        mesh=mesh,
        compiler_params=pltpu.CompilerParams(use_tc_tiling_on_sc=False),
        out_type=jax.ShapeDtypeStruct((NC, L), jnp.float32),
        scratch_types=[
            pltpu.VMEM((4, CH), jnp.int32),
            pltpu.VMEM((4, CH), jnp.int32),
            pltpu.VMEM((4, CH), jnp.float32),
            pltpu.VMEM((4, CH), jnp.float32),
            pltpu.VMEM((2, CH, L), jnp.float32),
            pltpu.VMEM((2, CH, L), jnp.float32),
            pltpu.VMEM((L,), jnp.float32),
            pltpu.VMEM((L,), jnp.float32),
            pltpu.VMEM_SHARED((NS, L), jnp.float32),
            pltpu.VMEM_SHARED((3200 * NS, L), jnp.float32),
            pltpu.SemaphoreType.DMA,
            pltpu.SemaphoreType.DMA,
            pltpu.SemaphoreType.DMA,
            pltpu.SemaphoreType.DMA,
            pltpu.SemaphoreType.DMA,
            pltpu.SemaphoreType.DMA,
        ],
    )
    def k(table_hbm, row_hbm, col_hbm, wp_hbm, wn_hbm, num_out,
          idx_a, idx_b, wp_v, wn_v, rows_a, rows_b,
          acc_v, tmp_v, shared_num, shared_tab,
          sem_l0, sem_l1, sem_l2, sem_l3, sem_g0, sem_g1):
        c = lax.axis_index("c")
        s = lax.axis_index("s")
        wid = s * NC + c
        ebase = wid * per
        sem_l = (sem_l0, sem_l1, sem_l2, sem_l3)
        sem_g = (sem_g0, sem_g1)

        trows = shared_tab.shape[0] // NS
        toff = pl.multiple_of(s * trows, 8)
        pltpu.sync_copy(table_hbm.at[pl.ds(toff, trows)],
                        shared_tab.at[pl.ds(toff, trows)])
        plsc.subcore_barrier()

        def fire_lin(g, b):
            off = pl.multiple_of(ebase + g * CH, 8)
            pltpu.async_copy(row_hbm.at[pl.ds(off, CH)], idx_a.at[b], sem_l[b])
            pltpu.async_copy(col_hbm.at[pl.ds(off, CH)], idx_b.at[b], sem_l[b])
            pltpu.async_copy(wp_hbm.at[pl.ds(off, CH)], wp_v.at[b], sem_l[b])
            pltpu.async_copy(wn_hbm.at[pl.ds(off, CH)], wn_v.at[b], sem_l[b])

        def wait_lin(b):
            pltpu.make_async_copy(row_hbm.at[pl.ds(0, CH)], idx_a.at[b],
                                  sem_l[b]).wait()
            pltpu.make_async_copy(col_hbm.at[pl.ds(0, CH)], idx_b.at[b],
                                  sem_l[b]).wait()
            pltpu.make_async_copy(wp_hbm.at[pl.ds(0, CH)], wp_v.at[b],
                                  sem_l[b]).wait()
            pltpu.make_async_copy(wn_hbm.at[pl.ds(0, CH)], wn_v.at[b],
                                  sem_l[b]).wait()

        def fire_gath(lb, rb):
            for j in range(NG):
                sl = pl.ds(j * GW, GW)
                pltpu.async_copy(shared_tab.at[idx_a.at[lb].at[sl]],
                                 rows_a.at[rb].at[sl], sem_g[rb])
                pltpu.async_copy(shared_tab.at[idx_b.at[lb].at[sl]],
                                 rows_b.at[rb].at[sl], sem_g[rb])

        def wait_gath(rb):
            for j in range(NG):
                sl = pl.ds(j * GW, GW)
                pltpu.make_async_copy(table_hbm.at[pl.ds(0, GW)],
                                      rows_a.at[rb].at[sl], sem_g[rb]).wait()
                pltpu.make_async_copy(table_hbm.at[pl.ds(0, GW)],
                                      rows_b.at[rb].at[sl], sem_g[rb]).wait()

        def compute(lb, rb, acc):
            ra = rows_a.at[rb]
            rbv = rows_b.at[rb]
            wpr = wp_v.at[lb]
            wnr = wn_v.at[lb]

            def group_body(gi, acc):
                eb = gi * L
                wp16 = wpr[pl.ds(eb, L)]
                wd16 = wnr[pl.ds(eb, L)] - wp16
                for j in range(L):
                    a = ra[eb + j]
                    b = rbv[eb + j]
                    wpe = jnp.full((L,), wp16[j], jnp.float32)
                    wde = jnp.full((L,), wd16[j], jnp.float32)
                    acc = acc + a * (wpe * a + wde * b)
                return acc

            return plsc.parallel_loop(0, CH // L, 1, unroll=2,
                                      carry=acc)(group_body)

        nq = nchunk // 4

        fire_lin(0, 0)
        fire_lin(1, 1)
        wait_lin(0)
        fire_gath(0, 0)

        def quad_body(i, acc):
            g0 = 4 * i
            wait_lin(1)
            fire_gath(1, 1)
            fire_lin(g0 + 2, 2)
            wait_gath(0)
            acc = compute(0, 0, acc)
            wait_lin(2)
            fire_gath(2, 0)
            fire_lin(g0 + 3, 3)
            wait_gath(1)
            acc = compute(1, 1, acc)
            wait_lin(3)
            fire_gath(3, 1)
            fire_lin(g0 + 4, 0)
            wait_gath(0)
            acc = compute(2, 0, acc)
            wait_lin(0)
            fire_gath(0, 0)
            fire_lin(jnp.minimum(g0 + 5, nchunk - 1), 1)
            wait_gath(1)
            acc = compute(3, 1, acc)
            return acc

        num_acc = lax.fori_loop(0, nq, quad_body, jnp.zeros((L,), jnp.float32))

        wait_gath(0)
        num_acc = compute(0, 0, num_acc)
        wait_lin(1)

        acc_v[...] = num_acc
        pltpu.sync_copy(acc_v, shared_num.at[s])
        plsc.subcore_barrier()

        @pl.when(s == 0)
        def _():
            num_tot = jnp.zeros((L,), jnp.float32)
            for j in range(NS):
                pltpu.sync_copy(shared_num.at[j], tmp_v)
                num_tot = num_tot + tmp_v[...]
            acc_v[...] = num_tot
            pltpu.sync_copy(acc_v, num_out.at[c])

    return k


def _split_body(ei_ref, row_ref, col_ref):
    x = ei_ref[...]
    row_ref[...] = jnp.reshape(x[0:1, :], row_ref.shape)
    col_ref[...] = jnp.reshape(x[1:2, :], col_ref.shape)


def _split_edges(edge_index, e):
    rows8 = -(-(e // 128) // 8) * 8
    br = 1008
    grid = -(-rows8 // br)
    rows8 = grid * br
    bl = br * 128
    rows2, cols2 = pl.pallas_call(
        _split_body,
        grid=(grid,),
        in_specs=[pl.BlockSpec((2, bl), lambda i: (0, i))],
        out_specs=[pl.BlockSpec((br, 128), lambda i: (i, 0)),
                   pl.BlockSpec((br, 128), lambda i: (i, 0))],
        out_shape=[jax.ShapeDtypeStruct((rows8, 128), jnp.int32),
                   jax.ShapeDtypeStruct((rows8, 128), jnp.int32)],
    )(edge_index)
    return rows2.reshape(-1), cols2.reshape(-1)


def _flatten_body(pt_ref, tab_ref, den_ref, *, bc, n):
    i = pl.program_id(0)
    x = pt_ref[...]
    kk = x.shape[0]
    colmask = (i * bc + lax.broadcasted_iota(jnp.int32, (kk, bc), 1)) < n
    xm = jnp.where(colmask, x, 0.0)
    xt = jnp.transpose(xm)
    xt3 = jnp.reshape(xt, (bc // 8, 8, kk))
    z = jnp.zeros((bc // 8, L - kk), jnp.float32)
    pieces = []
    for t in range(8):
        pieces.append(xt3[:, t, :])
        pieces.append(z)
    tab_ref[...] = jnp.concatenate(pieces, axis=1)
    d = jnp.sum(xm * xm, axis=1)
    d16 = jnp.concatenate([d, jnp.zeros((L - kk,), jnp.float32)])

    @pl.when(i == 0)
    def _():
        den_ref[...] = jnp.zeros((1, L), jnp.float32)

    den_ref[...] += jnp.reshape(d16, (1, L))


def _flatten_prob(prob):
    n, kk = prob.shape
    bc = 6400
    grid = -(-n // bc)
    n_pad = grid * bc
    pt = prob.T
    tab, den = pl.pallas_call(
        lambda a, b, c: _flatten_body(a, b, c, bc=bc, n=n),
        grid=(grid,),
        in_specs=[pl.BlockSpec((kk, bc), lambda i: (0, i))],
        out_specs=[pl.BlockSpec((bc * L // 128, 128), lambda i: (i, 0)),
                   pl.BlockSpec((1, L), lambda i: (0, 0))],
        out_shape=[jax.ShapeDtypeStruct((n_pad * L // 128, 128), jnp.float32),
                   jax.ShapeDtypeStruct((1, L), jnp.float32)],
    )(pt)
    return tab.reshape(n_pad, L), den


def _combine_body(num_ref, den_ref, out_ref):
    num = jnp.sum(num_ref[...], axis=0, keepdims=True)
    den = den_ref[...] + 1.0
    out_ref[...] = jnp.full((1, L), jnp.sum(num / den), jnp.float32)


def kernel(prob, edge_index, w_p, w_n):
    e = w_p.shape[0]
    nchunk = e // (NW * CH)
    assert nchunk * CH * NW == e and nchunk % 4 == 1

    row, col = _split_edges(edge_index, e)
    table, den = _flatten_prob(prob)

    num_parts = _sc_edge_kernel(nchunk)(table, row, col, w_p, w_n)

    out = pl.pallas_call(
        _combine_body,
        out_shape=jax.ShapeDtypeStruct((1, L), jnp.float32),
    )(num_parts, den)
    return out[0, 0:1]
